# Initial kernel scaffold; baseline (speedup 1.0000x reference)
import jax
import jax.numpy as jnp
from jax.experimental import pallas as pl


def kernel(a, table):
    raise NotImplementedError("write your pallas kernel here")


# speedup vs baseline: 2.4906x; 2.4906x over previous
import jax
import jax.numpy as jnp
from jax import lax
from jax.experimental import pallas as pl
from jax.experimental.pallas import tpu as pltpu
from jax.experimental.pallas import tpu_sc as plsc

_NUM_CORES = 2
_NUM_SUBCORES = 16
_NUM_WORKERS = _NUM_CORES * _NUM_SUBCORES
_CHUNK = 2048


def _gather_body(a_hbm, table_hbm, out_hbm, idx_v, rows_v, sem):
    c = lax.axis_index("c")
    s = lax.axis_index("s")
    wid = s * _NUM_CORES + c
    n = a_hbm.shape[0]
    n_per_w = n // _NUM_WORKERS
    n_chunks = n_per_w // _CHUNK
    base = wid * n_per_w

    def chunk_body(i, carry):
        off = base + i * _CHUNK
        pltpu.sync_copy(a_hbm.at[pl.ds(off, _CHUNK)], idx_v)
        pltpu.async_copy(table_hbm.at[idx_v], rows_v, sem).wait()
        pltpu.sync_copy(rows_v, out_hbm.at[pl.ds(off, _CHUNK)])
        return carry

    lax.fori_loop(0, n_chunks, chunk_body, 0)


def kernel(a, table):
    b, h = a.shape
    n = b * h
    d = table.shape[1]
    a_flat = a.reshape(n).astype(jnp.int32)
    mesh = plsc.VectorSubcoreMesh(core_axis_name="c", subcore_axis_name="s")
    out = pl.kernel(
        _gather_body,
        out_type=jax.ShapeDtypeStruct((n, d), table.dtype),
        mesh=mesh,
        scratch_types=[
            pltpu.VMEM((_CHUNK,), jnp.int32),
            pltpu.VMEM((_CHUNK, d), jnp.float32),
            pltpu.SemaphoreType.DMA,
        ],
        compiler_params=pltpu.CompilerParams(use_tc_tiling_on_sc=False),
    )(a_flat, table)
    return out.reshape(b, h, d)


# speedup vs baseline: 2.5333x; 1.0171x over previous
import jax
import jax.numpy as jnp
from jax import lax
from jax.experimental import pallas as pl
from jax.experimental.pallas import tpu as pltpu
from jax.experimental.pallas import tpu_sc as plsc

_NUM_CORES = 2
_NUM_SUBCORES = 16
_NUM_WORKERS = _NUM_CORES * _NUM_SUBCORES
_CHUNK = 2048
_NBUF = 2


def _gather_body(a_hbm, table_hbm, out_hbm,
                 idx0, idx1, rows0, rows1,
                 sem_i0, sem_i1, sem_g0, sem_g1, sem_o0, sem_o1):
    idx_v = (idx0, idx1)
    rows_v = (rows0, rows1)
    sem_i = (sem_i0, sem_i1)
    sem_g = (sem_g0, sem_g1)
    sem_o = (sem_o0, sem_o1)

    c = lax.axis_index("c")
    s = lax.axis_index("s")
    wid = s * _NUM_CORES + c
    n = a_hbm.shape[0]
    n_per_w = n // _NUM_WORKERS
    n_chunks = n_per_w // _CHUNK
    n_outer = n_chunks // _NBUF
    base = wid * n_per_w

    def idx_load(i, b):
        pltpu.async_copy(a_hbm.at[pl.ds(base + i * _CHUNK, _CHUNK)],
                         idx_v[b], sem_i[b])

    def wait_idx(b):
        pltpu.make_async_copy(a_hbm.at[pl.ds(base, _CHUNK)],
                              idx_v[b], sem_i[b]).wait()

    def gather_start(b):
        pltpu.async_copy(table_hbm.at[idx_v[b]], rows_v[b], sem_g[b])

    def wait_gather(b):
        pltpu.make_async_copy(table_hbm.at[idx_v[b]],
                              rows_v[b], sem_g[b]).wait()

    def store_start(i, b):
        pltpu.async_copy(rows_v[b],
                         out_hbm.at[pl.ds(base + i * _CHUNK, _CHUNK)],
                         sem_o[b])

    def wait_store(b):
        pltpu.make_async_copy(rows_v[b],
                              out_hbm.at[pl.ds(base, _CHUNK)],
                              sem_o[b]).wait()

    for b in range(_NBUF):
        idx_load(b, b)

    for b in range(_NBUF):
        wait_idx(b)
        gather_start(b)
        wait_gather(b)
        store_start(b, b)
        idx_load(b + _NBUF, b)

    def outer(io, carry):
        i0 = io * _NBUF
        for b in range(_NBUF):
            wait_idx(b)
            wait_store(b)
            gather_start(b)
            wait_gather(b)
            store_start(i0 + b, b)
            idx_load(i0 + b + _NBUF, b)
        return carry

    lax.fori_loop(1, n_outer - 1, outer, 0)

    i0 = (n_outer - 1) * _NBUF
    for b in range(_NBUF):
        wait_idx(b)
        wait_store(b)
        gather_start(b)
        wait_gather(b)
        store_start(i0 + b, b)

    for b in range(_NBUF):
        wait_store(b)


def kernel(a, table):
    b, h = a.shape
    n = b * h
    d = table.shape[1]
    a_flat = a.reshape(n).astype(jnp.int32)
    mesh = plsc.VectorSubcoreMesh(core_axis_name="c", subcore_axis_name="s")
    out = pl.kernel(
        _gather_body,
        out_type=jax.ShapeDtypeStruct((n, d), table.dtype),
        mesh=mesh,
        scratch_types=(
            [pltpu.VMEM((_CHUNK,), jnp.int32) for _ in range(_NBUF)]
            + [pltpu.VMEM((_CHUNK, d), jnp.float32) for _ in range(_NBUF)]
            + [pltpu.SemaphoreType.DMA for _ in range(3 * _NBUF)]
        ),
        compiler_params=pltpu.CompilerParams(use_tc_tiling_on_sc=False),
    )(a_flat, table)
    return out.reshape(b, h, d)


# speedup vs baseline: 4.2533x; 1.6790x over previous
import jax
import jax.numpy as jnp
from jax import lax
from jax.experimental import pallas as pl
from jax.experimental.pallas import tpu as pltpu
from jax.experimental.pallas import tpu_sc as plsc

_NUM_CORES = 2
_NUM_SUBCORES = 16
_NUM_WORKERS = _NUM_CORES * _NUM_SUBCORES
_CHUNK = 512
_TILES = _CHUNK // 128


def _gather_body(a_hbm, table_hbm, out_hbm,
                 idx0, idx1, rows0, rows1, xb0, xb1,
                 sem_i0, sem_i1, sem_g0, sem_g1, sem_o0, sem_o1):
    idx_v = (idx0, idx1)
    rows_v = (rows0, rows1)
    xbuf = (xb0, xb1)
    sem_i = (sem_i0, sem_i1)
    sem_g = (sem_g0, sem_g1)
    sem_o = (sem_o0, sem_o1)

    c = lax.axis_index("c")
    s = lax.axis_index("s")
    wid = s * _NUM_CORES + c
    n = a_hbm.shape[0]
    batch = n // 200
    n_chunks = n // (_NUM_WORKERS * _CHUNK)
    h_stride = 16 * batch
    tc_stride = 8 * batch
    w_off = wid * (_TILES * 1024)

    lane = lax.iota(jnp.int32, 16)
    scat_p = (lane // 8) * (_TILES * 1024) + (lane % 8) * 128

    def idx_load(i, p):
        pltpu.async_copy(
            a_hbm.at[pl.ds(i * batch + wid * _CHUNK, _CHUNK)],
            idx_v[p], sem_i[p])

    def wait_idx(p):
        pltpu.make_async_copy(a_hbm.at[pl.ds(0, _CHUNK)],
                              idx_v[p], sem_i[p]).wait()

    def gather_start(p):
        pltpu.async_copy(table_hbm.at[idx_v[p]], rows_v[p], sem_g[p])

    def wait_gather(p):
        pltpu.make_async_copy(table_hbm.at[idx_v[p]],
                              rows_v[p], sem_g[p]).wait()

    def transpose(p):
        def t_body(t, carry):
            toff = t * 1024
            for bi in range(128):
                row = rows_v[p][t * 128 + bi, :]
                plsc.store_scatter(xbuf[p], [scat_p + (toff + bi)], row)
            return carry
        lax.fori_loop(0, _TILES, t_body, 0)

    def store_outs(i, p):
        base = i * h_stride + w_off
        pltpu.async_copy(xbuf[p].at[pl.ds(0, _TILES * 1024)],
                         out_hbm.at[pl.ds(base, _TILES * 1024)], sem_o[p])
        pltpu.async_copy(xbuf[p].at[pl.ds(_TILES * 1024, _TILES * 1024)],
                         out_hbm.at[pl.ds(base + tc_stride, _TILES * 1024)],
                         sem_o[p])

    def wait_outs(p):
        pltpu.make_async_copy(out_hbm.at[pl.ds(0, 2 * _TILES * 1024)],
                              xbuf[p], sem_o[p]).wait()

    for p in range(2):
        idx_load(p, p)
    for p in range(2):
        wait_idx(p)
        gather_start(p)

    for p in range(2):
        wait_gather(p)
        idx_load(p + 2, p)
        transpose(p)
        store_outs(p, p)
        wait_idx(p)
        gather_start(p)

    def outer(io, carry):
        i0 = io * 2
        for p in range(2):
            wait_gather(p)
            idx_load(i0 + p + 2, p)
            wait_outs(p)
            transpose(p)
            store_outs(i0 + p, p)
            wait_idx(p)
            gather_start(p)
        return carry

    lax.fori_loop(1, n_chunks // 2 - 1, outer, 0)

    i0 = n_chunks - 2
    for p in range(2):
        wait_gather(p)
        wait_outs(p)
        transpose(p)
        store_outs(i0 + p, p)

    for p in range(2):
        wait_outs(p)


def kernel(a, table):
    b, h = a.shape
    n = b * h
    d = table.shape[1]
    a_flat = a.T.reshape(n).astype(jnp.int32)
    mesh = plsc.VectorSubcoreMesh(core_axis_name="c", subcore_axis_name="s")
    out = pl.kernel(
        _gather_body,
        out_type=jax.ShapeDtypeStruct((n * d,), table.dtype),
        mesh=mesh,
        scratch_types=(
            [pltpu.VMEM((_CHUNK,), jnp.int32) for _ in range(2)]
            + [pltpu.VMEM((_CHUNK, d), jnp.float32) for _ in range(2)]
            + [pltpu.VMEM((2 * _TILES * 1024,), jnp.float32) for _ in range(2)]
            + [pltpu.SemaphoreType.DMA for _ in range(6)]
        ),
        compiler_params=pltpu.CompilerParams(
            use_tc_tiling_on_sc=False, needs_layout_passes=False),
    )(a_flat, table)
    x5 = out.reshape(h, 2, b // 128, 8, 128)
    return x5.transpose(2, 4, 0, 1, 3).reshape(b, h, d)


# speedup vs baseline: 5.0615x; 1.1900x over previous
import jax
import jax.numpy as jnp
from jax import lax
from jax.experimental import pallas as pl
from jax.experimental.pallas import tpu as pltpu
from jax.experimental.pallas import tpu_sc as plsc

_NUM_CORES = 2
_NUM_SUBCORES = 16
_NUM_WORKERS = _NUM_CORES * _NUM_SUBCORES
_CHUNK = 512
_TILES = _CHUNK // 128


def _gather_body(a_hbm, table_hbm, out_hbm,
                 idx0, idx1, rows0, rows1, xb0, xb1,
                 sem_i0, sem_i1, sem_g0, sem_g1, sem_o0, sem_o1):
    idx_v = (idx0, idx1)
    rows_v = (rows0, rows1)
    xbuf = (xb0, xb1)
    sem_i = (sem_i0, sem_i1)
    sem_g = (sem_g0, sem_g1)
    sem_o = (sem_o0, sem_o1)

    c = lax.axis_index("c")
    s = lax.axis_index("s")
    wid = s * _NUM_CORES + c
    n = a_hbm.shape[0]
    batch = n // 200
    n_chunks = n // (_NUM_WORKERS * _CHUNK)
    h_stride = 16 * batch
    tc_stride = 8 * batch
    w_off = wid * (_TILES * 1024)

    lane = lax.iota(jnp.int32, 16)
    col_ids = [lane * 0 + cc for cc in range(16)]

    def idx_load(i, p):
        pltpu.async_copy(
            a_hbm.at[pl.ds(i * batch + wid * _CHUNK, _CHUNK)],
            idx_v[p], sem_i[p])

    def wait_idx(p):
        pltpu.make_async_copy(a_hbm.at[pl.ds(0, _CHUNK)],
                              idx_v[p], sem_i[p]).wait()

    def gather_start(p):
        pltpu.async_copy(table_hbm.at[idx_v[p]], rows_v[p], sem_g[p])

    def wait_gather(p):
        pltpu.make_async_copy(table_hbm.at[idx_v[p]],
                              rows_v[p], sem_g[p]).wait()

    def transpose(p):
        def r_body(r0, carry):
            r = r0 * 16
            rowidx = lane + r
            off = (r // 128) * 1024 + (r % 128)
            for cc in range(16):
                vec = plsc.load_gather(rows_v[p], [rowidx, col_ids[cc]])
                base_c = (cc // 8) * (_TILES * 1024) + (cc % 8) * 128
                xbuf[p][pl.ds(base_c + off, 16)] = vec
            return carry
        lax.fori_loop(0, _CHUNK // 16, r_body, 0)

    def store_outs(i, p):
        base = i * h_stride + w_off
        pltpu.async_copy(xbuf[p].at[pl.ds(0, _TILES * 1024)],
                         out_hbm.at[pl.ds(base, _TILES * 1024)], sem_o[p])
        pltpu.async_copy(xbuf[p].at[pl.ds(_TILES * 1024, _TILES * 1024)],
                         out_hbm.at[pl.ds(base + tc_stride, _TILES * 1024)],
                         sem_o[p])

    def wait_outs(p):
        pltpu.make_async_copy(out_hbm.at[pl.ds(0, 2 * _TILES * 1024)],
                              xbuf[p], sem_o[p]).wait()

    for p in range(2):
        idx_load(p, p)
    for p in range(2):
        wait_idx(p)
        gather_start(p)

    for p in range(2):
        wait_gather(p)
        idx_load(p + 2, p)
        transpose(p)
        store_outs(p, p)
        wait_idx(p)
        gather_start(p)

    def outer(io, carry):
        i0 = io * 2
        for p in range(2):
            wait_gather(p)
            idx_load(i0 + p + 2, p)
            wait_outs(p)
            transpose(p)
            store_outs(i0 + p, p)
            wait_idx(p)
            gather_start(p)
        return carry

    lax.fori_loop(1, n_chunks // 2 - 1, outer, 0)

    i0 = n_chunks - 2
    for p in range(2):
        wait_gather(p)
        wait_outs(p)
        transpose(p)
        store_outs(i0 + p, p)

    for p in range(2):
        wait_outs(p)


def kernel(a, table):
    b, h = a.shape
    n = b * h
    d = table.shape[1]
    a_flat = a.T.reshape(n).astype(jnp.int32)
    mesh = plsc.VectorSubcoreMesh(core_axis_name="c", subcore_axis_name="s")
    out = pl.kernel(
        _gather_body,
        out_type=jax.ShapeDtypeStruct((n * d,), table.dtype),
        mesh=mesh,
        scratch_types=(
            [pltpu.VMEM((_CHUNK,), jnp.int32) for _ in range(2)]
            + [pltpu.VMEM((_CHUNK, d), jnp.float32) for _ in range(2)]
            + [pltpu.VMEM((2 * _TILES * 1024,), jnp.float32) for _ in range(2)]
            + [pltpu.SemaphoreType.DMA for _ in range(6)]
        ),
        compiler_params=pltpu.CompilerParams(
            use_tc_tiling_on_sc=False, needs_layout_passes=False),
    )(a_flat, table)
    x5 = out.reshape(h, 2, b // 128, 8, 128)
    return x5.transpose(2, 4, 0, 1, 3).reshape(b, h, d)


# speedup vs baseline: 7.0905x; 1.4009x over previous
import jax
import jax.numpy as jnp
from jax import lax
from jax.experimental import pallas as pl
from jax.experimental.pallas import tpu as pltpu
from jax.experimental.pallas import tpu_sc as plsc

_NUM_CORES = 2
_NUM_SUBCORES = 16
_NUM_WORKERS = _NUM_CORES * _NUM_SUBCORES
_CHUNK = 512
_TILES = _CHUNK // 128


def _gather_body(a_hbm, table_hbm, out_hbm,
                 idx0, idx1, rows0, rows1, xb0, xb1,
                 sem_i0, sem_i1, sem_g0, sem_g1, sem_o0, sem_o1):
    idx_v = (idx0, idx1)
    rows_v = (rows0, rows1)
    xbuf = (xb0, xb1)
    sem_i = (sem_i0, sem_i1)
    sem_g = (sem_g0, sem_g1)
    sem_o = (sem_o0, sem_o1)

    c = lax.axis_index("c")
    s = lax.axis_index("s")
    wid = s * _NUM_CORES + c
    n = a_hbm.shape[0]
    batch = n // 200
    n_chunks = n // (_NUM_WORKERS * _CHUNK)
    h_stride = 16 * batch
    tc_stride = 8 * batch
    w_off = wid * (_TILES * 1024)

    lane = lax.iota(jnp.int32, 16)
    cmods = [(lane + d) % 16 for d in range(16)]
    scat_q = [((cm // 8) * (_TILES * 1024) + (cm % 8) * 128 + lane)
              for cm in cmods]

    def idx_load(i, p):
        pltpu.async_copy(
            a_hbm.at[pl.ds(i * batch + wid * _CHUNK, _CHUNK)],
            idx_v[p], sem_i[p])

    def wait_idx(p):
        pltpu.make_async_copy(a_hbm.at[pl.ds(0, _CHUNK)],
                              idx_v[p], sem_i[p]).wait()

    def gather_start(p):
        pltpu.async_copy(table_hbm.at[idx_v[p]], rows_v[p], sem_g[p])

    def wait_gather(p):
        pltpu.make_async_copy(table_hbm.at[idx_v[p]],
                              rows_v[p], sem_g[p]).wait()

    def transpose(p):
        def r_body(r0, carry):
            rowidx = lane + r0 * 16
            soff = (r0 // 8) * 1024 + (r0 % 8) * 16
            for d in range(16):
                vec = plsc.load_gather(rows_v[p], [rowidx, cmods[d]])
                plsc.store_scatter(xbuf[p], [scat_q[d] + soff], vec)
            return carry
        lax.fori_loop(0, _CHUNK // 16, r_body, 0)

    def store_outs(i, p):
        base = i * h_stride + w_off
        pltpu.async_copy(xbuf[p].at[pl.ds(0, _TILES * 1024)],
                         out_hbm.at[pl.ds(base, _TILES * 1024)], sem_o[p])
        pltpu.async_copy(xbuf[p].at[pl.ds(_TILES * 1024, _TILES * 1024)],
                         out_hbm.at[pl.ds(base + tc_stride, _TILES * 1024)],
                         sem_o[p])

    def wait_outs(p):
        pltpu.make_async_copy(out_hbm.at[pl.ds(0, 2 * _TILES * 1024)],
                              xbuf[p], sem_o[p]).wait()

    for p in range(2):
        idx_load(p, p)
    for p in range(2):
        wait_idx(p)
        gather_start(p)

    for p in range(2):
        wait_gather(p)
        idx_load(p + 2, p)
        transpose(p)
        store_outs(p, p)
        wait_idx(p)
        gather_start(p)

    def outer(io, carry):
        i0 = io * 2
        for p in range(2):
            wait_gather(p)
            idx_load(i0 + p + 2, p)
            wait_outs(p)
            transpose(p)
            store_outs(i0 + p, p)
            wait_idx(p)
            gather_start(p)
        return carry

    lax.fori_loop(1, n_chunks // 2 - 1, outer, 0)

    i0 = n_chunks - 2
    for p in range(2):
        wait_gather(p)
        wait_outs(p)
        transpose(p)
        store_outs(i0 + p, p)

    for p in range(2):
        wait_outs(p)


def kernel(a, table):
    b, h = a.shape
    n = b * h
    d = table.shape[1]
    a_flat = a.T.reshape(n).astype(jnp.int32)
    mesh = plsc.VectorSubcoreMesh(core_axis_name="c", subcore_axis_name="s")
    out = pl.kernel(
        _gather_body,
        out_type=jax.ShapeDtypeStruct((n * d,), table.dtype),
        mesh=mesh,
        scratch_types=(
            [pltpu.VMEM((_CHUNK,), jnp.int32) for _ in range(2)]
            + [pltpu.VMEM((_CHUNK, d), jnp.float32) for _ in range(2)]
            + [pltpu.VMEM((2 * _TILES * 1024,), jnp.float32) for _ in range(2)]
            + [pltpu.SemaphoreType.DMA for _ in range(6)]
        ),
        compiler_params=pltpu.CompilerParams(
            use_tc_tiling_on_sc=False, needs_layout_passes=False),
    )(a_flat, table)
    x5 = out.reshape(h, 2, b // 128, 8, 128)
    return x5.transpose(2, 4, 0, 1, 3).reshape(b, h, d)


# speedup vs baseline: 8.8735x; 1.2515x over previous
import jax
import jax.numpy as jnp
from jax import lax
from jax.experimental import pallas as pl
from jax.experimental.pallas import tpu as pltpu
from jax.experimental.pallas import tpu_sc as plsc

_NUM_CORES = 2
_NUM_SUBCORES = 16
_NUM_WORKERS = _NUM_CORES * _NUM_SUBCORES
_CHUNK = 512
_TILES = _CHUNK // 128


def _gather_body(a_hbm, table_hbm, out_hbm,
                 idx0, idx1, rows0, rows1, xb0, xb1,
                 sem_i0, sem_i1, sem_g0, sem_g1, sem_o0, sem_o1):
    idx_v = (idx0, idx1)
    rows_v = (rows0, rows1)
    xbuf = (xb0, xb1)
    sem_i = (sem_i0, sem_i1)
    sem_g = (sem_g0, sem_g1)
    sem_o = (sem_o0, sem_o1)

    c = lax.axis_index("c")
    s = lax.axis_index("s")
    wid = s * _NUM_CORES + c
    n = a_hbm.shape[0]
    batch = n // 200
    n_chunks = n // (_NUM_WORKERS * _CHUNK)
    h_stride = 16 * batch
    tc_stride = 8 * batch
    w_off = wid * (_TILES * 1024)

    lane = lax.iota(jnp.int32, 16)
    cmods = [(lane + d) % 16 for d in range(16)]
    scat_q = [((cm // 8) * (_TILES * 1024) + (cm % 8) * 128 + lane)
              for cm in cmods]

    def idx_load(i, p):
        pltpu.async_copy(
            a_hbm.at[pl.ds(i * batch + wid * _CHUNK, _CHUNK)],
            idx_v[p], sem_i[p])

    def wait_idx(p):
        pltpu.make_async_copy(a_hbm.at[pl.ds(0, _CHUNK)],
                              idx_v[p], sem_i[p]).wait()

    def gather_start(p):
        pltpu.async_copy(table_hbm.at[idx_v[p]], rows_v[p], sem_g[p])

    def wait_gather(p):
        pltpu.make_async_copy(table_hbm.at[idx_v[p]],
                              rows_v[p], sem_g[p]).wait()

    def transpose(p):
        @plsc.parallel_loop(0, _CHUNK // 16, step=1, unroll=2)
        def r_body(r0):
            rowidx = lane + r0 * 16
            soff = (r0 // 8) * 1024 + (r0 % 8) * 16
            for d in range(16):
                vec = plsc.load_gather(rows_v[p], [rowidx, cmods[d]])
                plsc.store_scatter(xbuf[p], [scat_q[d] + soff], vec)

    def store_outs(i, p):
        base = i * h_stride + w_off
        pltpu.async_copy(xbuf[p].at[pl.ds(0, _TILES * 1024)],
                         out_hbm.at[pl.ds(base, _TILES * 1024)], sem_o[p])
        pltpu.async_copy(xbuf[p].at[pl.ds(_TILES * 1024, _TILES * 1024)],
                         out_hbm.at[pl.ds(base + tc_stride, _TILES * 1024)],
                         sem_o[p])

    def wait_outs(p):
        pltpu.make_async_copy(out_hbm.at[pl.ds(0, 2 * _TILES * 1024)],
                              xbuf[p], sem_o[p]).wait()

    for p in range(2):
        idx_load(p, p)
    for p in range(2):
        wait_idx(p)
        gather_start(p)

    for p in range(2):
        wait_gather(p)
        idx_load(p + 2, p)
        transpose(p)
        store_outs(p, p)
        wait_idx(p)
        gather_start(p)

    def outer(io, carry):
        i0 = io * 2
        for p in range(2):
            wait_gather(p)
            idx_load(i0 + p + 2, p)
            wait_outs(p)
            transpose(p)
            store_outs(i0 + p, p)
            wait_idx(p)
            gather_start(p)
        return carry

    lax.fori_loop(1, n_chunks // 2 - 1, outer, 0)

    i0 = n_chunks - 2
    for p in range(2):
        wait_gather(p)
        wait_outs(p)
        transpose(p)
        store_outs(i0 + p, p)

    for p in range(2):
        wait_outs(p)


def kernel(a, table):
    b, h = a.shape
    n = b * h
    d = table.shape[1]
    a_flat = a.T.reshape(n).astype(jnp.int32)
    mesh = plsc.VectorSubcoreMesh(core_axis_name="c", subcore_axis_name="s")
    out = pl.kernel(
        _gather_body,
        out_type=jax.ShapeDtypeStruct((n * d,), table.dtype),
        mesh=mesh,
        scratch_types=(
            [pltpu.VMEM((_CHUNK,), jnp.int32) for _ in range(2)]
            + [pltpu.VMEM((_CHUNK, d), jnp.float32) for _ in range(2)]
            + [pltpu.VMEM((2 * _TILES * 1024,), jnp.float32) for _ in range(2)]
            + [pltpu.SemaphoreType.DMA for _ in range(6)]
        ),
        compiler_params=pltpu.CompilerParams(
            use_tc_tiling_on_sc=False, needs_layout_passes=False),
    )(a_flat, table)
    x5 = out.reshape(h, 2, b // 128, 8, 128)
    return x5.transpose(2, 4, 0, 1, 3).reshape(b, h, d)
